# baseline (speedup 1.0000x reference)
import math

import jax
import jax.numpy as jnp
from jax import lax
from jax.experimental import pallas as pl
from jax.experimental.pallas import tpu as pltpu

D = 128
H = 128
B = 16
T = 512
STEPS = T // B
TOTAL = STEPS * (B * (B + 1)) // 2


def _rnn_kernel(data_ref, h0_ref, c0_ref, wih_t_ref, whh_t_ref,
                bih_ref, bhh_ref, out_ref, h_ref, c_ref, xp_ref):
    bias = bih_ref[...] + bhh_ref[...]
    wih_t = wih_t_ref[...]

    def proj_body(i, _):
        off = pl.multiple_of(i * 64, 8)
        xp_ref[pl.ds(off, 64), :] = jnp.dot(
            data_ref[pl.ds(off, 64), :], wih_t,
            preferred_element_type=jnp.float32) + bias
        return 0

    lax.fori_loop(0, TOTAL // 64, proj_body, 0)

    whh_t = whh_t_ref[...]
    hA = h0_ref[0:8, :]
    cA = c0_ref[0:8, :]
    hB = h0_ref[8:16, :]
    cB = c0_ref[8:16, :]

    def sig(x):
        return 0.5 + 0.5 * jnp.tanh(0.5 * x)

    def cell(x, hb, cb):
        g = x + jnp.dot(hb, whh_t, preferred_element_type=jnp.float32)
        ig = sig(g[:, 0:H])
        fg = sig(g[:, H:2 * H])
        gg = jnp.tanh(g[:, 2 * H:3 * H])
        og = sig(g[:, 3 * H:4 * H])
        c_new = fg * cb + ig * gg
        h_new = og * jnp.tanh(c_new)
        return h_new, c_new

    row = lax.broadcasted_iota(jnp.int32, (8, H), 0)

    for p in range(B):
        bs = p + 1
        aA = min(bs, 8)
        aB = bs - aA
        grp = math.lcm(bs, 8) // bs
        blk = grp * bs
        ngroups = STEPS // grp
        phase_end = TOTAL - STEPS * (p * (p + 1) // 2)
        mA = row < aA
        mB = row < aB

        def gbody(j, carry, bs=bs, aA=aA, aB=aB, grp=grp, blk=blk,
                  phase_end=phase_end, mA=mA, mB=mB):
            hA, cA, hB, cB = carry
            goff = pl.multiple_of(phase_end - (j + 1) * blk, 8)
            xg = xp_ref[pl.ds(goff, blk), :]
            outs = [None] * grp
            for m in range(grp):
                s = grp - 1 - m
                xa = lax.slice(xg, (s * bs, 0), (s * bs + aA, 4 * H))
                if aA < 8:
                    xa = jnp.concatenate(
                        [xa, jnp.zeros((8 - aA, 4 * H), jnp.float32)], axis=0)
                ha_new, ca_new = cell(xa, hA, cA)
                if aA < 8:
                    hA = jnp.where(mA, ha_new, hA)
                    cA = jnp.where(mA, ca_new, cA)
                else:
                    hA, cA = ha_new, ca_new
                if aB > 0:
                    xb = lax.slice(xg, (s * bs + 8, 0), (s * bs + bs, 4 * H))
                    if aB < 8:
                        xb = jnp.concatenate(
                            [xb, jnp.zeros((8 - aB, 4 * H), jnp.float32)],
                            axis=0)
                    hb_new, cb_new = cell(xb, hB, cB)
                    if aB < 8:
                        hB = jnp.where(mB, hb_new, hB)
                        cB = jnp.where(mB, cb_new, cB)
                    else:
                        hB, cB = hb_new, cb_new
                    outs[s] = jnp.concatenate(
                        [hA[0:aA, :], hB[0:aB, :]], axis=0)
                else:
                    outs[s] = hA[0:aA, :]
            block = outs[0] if grp == 1 else jnp.concatenate(outs, axis=0)
            out_ref[pl.ds(goff, blk), :] = block
            return hA, cA, hB, cB

        hA, cA, hB, cB = lax.fori_loop(0, ngroups, gbody, (hA, cA, hB, cB))

    h_ref[...] = jnp.concatenate([hA, hB], axis=0)
    c_ref[...] = jnp.concatenate([cA, cB], axis=0)


def kernel(data, batch_sizes, h0, c0, W_ih, W_hh, b_ih, b_hh):
    del batch_sizes
    out, h, c = pl.pallas_call(
        _rnn_kernel,
        out_shape=(
            jax.ShapeDtypeStruct((TOTAL, H), jnp.float32),
            jax.ShapeDtypeStruct((B, H), jnp.float32),
            jax.ShapeDtypeStruct((B, H), jnp.float32),
        ),
        in_specs=[pl.BlockSpec(memory_space=pltpu.VMEM)] * 7,
        out_specs=(
            pl.BlockSpec(memory_space=pltpu.VMEM),
            pl.BlockSpec(memory_space=pltpu.VMEM),
            pl.BlockSpec(memory_space=pltpu.VMEM),
        ),
        scratch_shapes=[pltpu.VMEM((TOTAL, 4 * H), jnp.float32)],
    )(data, h0, c0, W_ih.T, W_hh.T,
      b_ih.reshape(1, 4 * H), b_hh.reshape(1, 4 * H))
    return out, h, c


# speedup vs baseline: 1.0649x; 1.0649x over previous
import math

import jax
import jax.numpy as jnp
from jax import lax
from jax.experimental import pallas as pl
from jax.experimental.pallas import tpu as pltpu

D = 128
H = 128
B = 16
T = 512
STEPS = T // B
TOTAL = STEPS * (B * (B + 1)) // 2


def _rnn_kernel(data_ref, h0_ref, c0_ref, wih_t_ref, whh_t_ref,
                bih_ref, bhh_ref, out_ref, h_ref, c_ref, xp_ref):
    bias = bih_ref[...] + bhh_ref[...]
    wih_t = wih_t_ref[...]

    def proj_body(i, _):
        off = pl.multiple_of(i * 128, 8)
        xp_ref[pl.ds(off, 128), :] = jnp.dot(
            data_ref[pl.ds(off, 128), :], wih_t,
            preferred_element_type=jnp.float32) + bias
        return 0

    lax.fori_loop(0, TOTAL // 128, proj_body, 0)

    whh_bf = whh_t_ref[...].astype(jnp.bfloat16)
    hA = h0_ref[0:8, :]
    cA = c0_ref[0:8, :]
    hB = h0_ref[8:16, :]
    cB = c0_ref[8:16, :]

    def rdot(hb):
        return jnp.dot(hb.astype(jnp.bfloat16), whh_bf,
                       preferred_element_type=jnp.float32)

    def acts(x, d, cb):
        raw = jnp.tanh(x + d)
        ig = 0.5 + 0.5 * raw[:, 0:H]
        fg = 0.5 + 0.5 * raw[:, H:2 * H]
        gg = raw[:, 2 * H:3 * H]
        og = 0.5 + 0.5 * raw[:, 3 * H:4 * H]
        c_new = fg * cb + ig * gg
        h_new = og * jnp.tanh(c_new)
        return h_new, c_new

    row = lax.broadcasted_iota(jnp.int32, (8, H), 0)

    for p in range(B):
        bs = p + 1
        aA = min(bs, 8)
        aB = bs - aA
        grp = math.lcm(bs, 8) // bs
        blk = grp * bs
        ngroups = STEPS // grp
        phase_end = TOTAL - STEPS * (p * (p + 1) // 2)
        mA = row < aA
        mB = row < aB

        ug = max(1, 16 // grp)
        nit = ngroups // ug

        def gbody(j, carry, bs=bs, aA=aA, aB=aB, grp=grp, blk=blk, ug=ug,
                  phase_end=phase_end, mA=mA, mB=mB):
            hA, cA, hB, cB = carry
            for t in range(ug):
                goff = pl.multiple_of(
                    phase_end - (j * ug + t + 1) * blk, 8)
                xg = xp_ref[pl.ds(goff, blk), :]
                outs = [None] * grp
                for m in range(grp):
                    s = grp - 1 - m
                    xa = lax.slice(xg, (s * bs, 0), (s * bs + aA, 4 * H))
                    if aA < 8:
                        xa = jnp.concatenate(
                            [xa, jnp.zeros((8 - aA, 4 * H), jnp.float32)],
                            axis=0)
                    dA = rdot(hA)
                    if aB > 0:
                        xb = lax.slice(xg, (s * bs + 8, 0),
                                       (s * bs + bs, 4 * H))
                        if aB < 8:
                            xb = jnp.concatenate(
                                [xb, jnp.zeros((8 - aB, 4 * H), jnp.float32)],
                                axis=0)
                        dB = rdot(hB)
                    ha_new, ca_new = acts(xa, dA, cA)
                    if aA < 8:
                        hA = jnp.where(mA, ha_new, hA)
                        cA = jnp.where(mA, ca_new, cA)
                    else:
                        hA, cA = ha_new, ca_new
                    if aB > 0:
                        hb_new, cb_new = acts(xb, dB, cB)
                        if aB < 8:
                            hB = jnp.where(mB, hb_new, hB)
                            cB = jnp.where(mB, cb_new, cB)
                        else:
                            hB, cB = hb_new, cb_new
                        outs[s] = jnp.concatenate(
                            [hA[0:aA, :], hB[0:aB, :]], axis=0)
                    else:
                        outs[s] = hA[0:aA, :]
                block = outs[0] if grp == 1 else jnp.concatenate(outs, axis=0)
                out_ref[pl.ds(goff, blk), :] = block
            return hA, cA, hB, cB

        hA, cA, hB, cB = lax.fori_loop(0, nit, gbody, (hA, cA, hB, cB))

    h_ref[...] = jnp.concatenate([hA, hB], axis=0)
    c_ref[...] = jnp.concatenate([cA, cB], axis=0)


def kernel(data, batch_sizes, h0, c0, W_ih, W_hh, b_ih, b_hh):
    del batch_sizes
    scale = jnp.concatenate([jnp.full((2 * H,), 0.5, jnp.float32),
                             jnp.ones((H,), jnp.float32),
                             jnp.full((H,), 0.5, jnp.float32)])
    W_ih = W_ih * scale[:, None]
    W_hh = W_hh * scale[:, None]
    b_ih = b_ih * scale
    b_hh = b_hh * scale
    out, h, c = pl.pallas_call(
        _rnn_kernel,
        out_shape=(
            jax.ShapeDtypeStruct((TOTAL, H), jnp.float32),
            jax.ShapeDtypeStruct((B, H), jnp.float32),
            jax.ShapeDtypeStruct((B, H), jnp.float32),
        ),
        in_specs=[pl.BlockSpec(memory_space=pltpu.VMEM)] * 7,
        out_specs=(
            pl.BlockSpec(memory_space=pltpu.VMEM),
            pl.BlockSpec(memory_space=pltpu.VMEM),
            pl.BlockSpec(memory_space=pltpu.VMEM),
        ),
        scratch_shapes=[pltpu.VMEM((TOTAL, 4 * H), jnp.float32)],
    )(data, h0, c0, W_ih.T, W_hh.T,
      b_ih.reshape(1, 4 * H), b_hh.reshape(1, 4 * H))
    return out, h, c


# speedup vs baseline: 1.1104x; 1.0427x over previous
import math

import jax
import jax.numpy as jnp
from jax import lax
from jax.experimental import pallas as pl
from jax.experimental.pallas import tpu as pltpu

D = 128
H = 128
B = 16
T = 512
STEPS = T // B
TOTAL = STEPS * (B * (B + 1)) // 2


def _rnn_kernel(data_ref, h0_ref, c0_ref, wih_t_ref, whh_t_ref,
                bih_ref, bhh_ref, out_ref, h_ref, c_ref, xp_ref):
    bias = bih_ref[...] + bhh_ref[...]
    wih_t = wih_t_ref[...]

    def proj_body(i, _):
        for t in range(2):
            off = pl.multiple_of(i * 256 + t * 128, 8)
            xp_ref[pl.ds(off, 128), :] = jnp.dot(
                data_ref[pl.ds(off, 128), :], wih_t,
                preferred_element_type=jnp.float32) + bias
        return 0

    lax.fori_loop(0, TOTAL // 256, proj_body, 0)

    whh_t = whh_t_ref[...]
    hA = h0_ref[0:8, :]
    cA = c0_ref[0:8, :]
    hB = h0_ref[8:16, :]
    cB = c0_ref[8:16, :]

    def rdot(hb):
        return jnp.dot(hb, whh_t, preferred_element_type=jnp.float32)

    def acts(x, d, cb):
        raw = jnp.tanh(x + d)
        ig = 0.5 + 0.5 * raw[:, 0:H]
        fg = 0.5 + 0.5 * raw[:, H:2 * H]
        gg = raw[:, 2 * H:3 * H]
        og = 0.5 + 0.5 * raw[:, 3 * H:4 * H]
        c_new = fg * cb + ig * gg
        h_new = og * jnp.tanh(c_new)
        return h_new, c_new

    row = lax.broadcasted_iota(jnp.int32, (8, H), 0)

    for p in range(B):
        bs = p + 1
        aA = min(bs, 8)
        aB = bs - aA
        grp = math.lcm(bs, 8) // bs
        blk = grp * bs
        ngroups = STEPS // grp
        phase_end = TOTAL - STEPS * (p * (p + 1) // 2)
        mA = row < aA
        mB = row < aB

        ug = max(1, 16 // grp)
        nit = ngroups // ug

        def gbody(j, carry, bs=bs, aA=aA, aB=aB, grp=grp, blk=blk, ug=ug,
                  phase_end=phase_end, mA=mA, mB=mB):
            hA, cA, hB, cB = carry
            for t in range(ug):
                goff = pl.multiple_of(
                    phase_end - (j * ug + t + 1) * blk, 8)
                xg = xp_ref[pl.ds(goff, blk), :]
                outs = [None] * grp
                for m in range(grp):
                    s = grp - 1 - m
                    xa = lax.slice(xg, (s * bs, 0), (s * bs + aA, 4 * H))
                    if aA < 8:
                        xa = jnp.concatenate(
                            [xa, jnp.zeros((8 - aA, 4 * H), jnp.float32)],
                            axis=0)
                    dA = rdot(hA)
                    if aB > 0:
                        xb = lax.slice(xg, (s * bs + 8, 0),
                                       (s * bs + bs, 4 * H))
                        if aB < 8:
                            xb = jnp.concatenate(
                                [xb, jnp.zeros((8 - aB, 4 * H), jnp.float32)],
                                axis=0)
                        dB = rdot(hB)
                    ha_new, ca_new = acts(xa, dA, cA)
                    if aA < 8:
                        hA = jnp.where(mA, ha_new, hA)
                        cA = jnp.where(mA, ca_new, cA)
                    else:
                        hA, cA = ha_new, ca_new
                    if aB > 0:
                        hb_new, cb_new = acts(xb, dB, cB)
                        if aB < 8:
                            hB = jnp.where(mB, hb_new, hB)
                            cB = jnp.where(mB, cb_new, cB)
                        else:
                            hB, cB = hb_new, cb_new
                        outs[s] = jnp.concatenate(
                            [hA[0:aA, :], hB[0:aB, :]], axis=0)
                    else:
                        outs[s] = hA[0:aA, :]
                block = outs[0] if grp == 1 else jnp.concatenate(outs, axis=0)
                out_ref[pl.ds(goff, blk), :] = block
            return hA, cA, hB, cB

        hA, cA, hB, cB = lax.fori_loop(0, nit, gbody, (hA, cA, hB, cB))

    h_ref[...] = jnp.concatenate([hA, hB], axis=0)
    c_ref[...] = jnp.concatenate([cA, cB], axis=0)


def kernel(data, batch_sizes, h0, c0, W_ih, W_hh, b_ih, b_hh):
    del batch_sizes
    scale = jnp.concatenate([jnp.full((2 * H,), 0.5, jnp.float32),
                             jnp.ones((H,), jnp.float32),
                             jnp.full((H,), 0.5, jnp.float32)])
    W_ih = W_ih * scale[:, None]
    W_hh = W_hh * scale[:, None]
    b_ih = b_ih * scale
    b_hh = b_hh * scale
    out, h, c = pl.pallas_call(
        _rnn_kernel,
        out_shape=(
            jax.ShapeDtypeStruct((TOTAL, H), jnp.float32),
            jax.ShapeDtypeStruct((B, H), jnp.float32),
            jax.ShapeDtypeStruct((B, H), jnp.float32),
        ),
        in_specs=[pl.BlockSpec(memory_space=pltpu.VMEM)] * 7,
        out_specs=(
            pl.BlockSpec(memory_space=pltpu.VMEM),
            pl.BlockSpec(memory_space=pltpu.VMEM),
            pl.BlockSpec(memory_space=pltpu.VMEM),
        ),
        scratch_shapes=[pltpu.VMEM((TOTAL, 4 * H), jnp.float32)],
    )(data, h0, c0, W_ih.T, W_hh.T,
      b_ih.reshape(1, 4 * H), b_hh.reshape(1, 4 * H))
    return out, h, c


# speedup vs baseline: 1.1220x; 1.0105x over previous
import math

import jax
import jax.numpy as jnp
from jax import lax
from jax.experimental import pallas as pl
from jax.experimental.pallas import tpu as pltpu

D = 128
H = 128
B = 16
T = 512
STEPS = T // B
TOTAL = STEPS * (B * (B + 1)) // 2


def _rnn_kernel(data_ref, h0_ref, c0_ref, wih_t_ref, whh_t_ref,
                bih_ref, bhh_ref, out_ref, h_ref, c_ref, xp_ref):
    bias = bih_ref[...] + bhh_ref[...]
    wih_t = wih_t_ref[...]

    def proj_body(i, _):
        for t in range(2):
            off = pl.multiple_of(i * 256 + t * 128, 8)
            xp_ref[pl.ds(off, 128), :] = jnp.dot(
                data_ref[pl.ds(off, 128), :], wih_t,
                preferred_element_type=jnp.float32) + bias
        return 0

    lax.fori_loop(0, TOTAL // 256, proj_body, 0)

    whh_t = whh_t_ref[...]
    hA = h0_ref[0:8, :]
    cA = c0_ref[0:8, :]
    hB = h0_ref[8:16, :]
    cB = c0_ref[8:16, :]

    def rdot(hb):
        return jnp.dot(hb, whh_t, preferred_element_type=jnp.float32)

    def acts(x, d, cb):
        raw = jnp.tanh(x + d)
        ig = 0.5 + 0.5 * raw[:, 0:H]
        fg = 0.5 + 0.5 * raw[:, H:2 * H]
        gg = raw[:, 2 * H:3 * H]
        og = 0.5 + 0.5 * raw[:, 3 * H:4 * H]
        c_new = fg * cb + ig * gg
        h_new = og * jnp.tanh(c_new)
        return h_new, c_new

    row = lax.broadcasted_iota(jnp.int32, (8, H), 0)

    for p in range(B):
        bs = p + 1
        aA = min(bs, 8)
        aB = bs - aA
        grp = math.lcm(bs, 8) // bs
        blk = grp * bs
        ngroups = STEPS // grp
        phase_end = TOTAL - STEPS * (p * (p + 1) // 2)
        mA = row < aA
        mB = row < aB

        ug = max(1, 32 // grp)
        nit = ngroups // ug

        def gbody(j, carry, bs=bs, aA=aA, aB=aB, grp=grp, blk=blk, ug=ug,
                  phase_end=phase_end, mA=mA, mB=mB):
            hA, cA, hB, cB = carry
            for t in range(ug):
                goff = pl.multiple_of(
                    phase_end - (j * ug + t + 1) * blk, 8)
                xg = xp_ref[pl.ds(goff, blk), :]
                outs = [None] * grp
                for m in range(grp):
                    s = grp - 1 - m
                    xa = lax.slice(xg, (s * bs, 0), (s * bs + aA, 4 * H))
                    if aA < 8:
                        xa = jnp.concatenate(
                            [xa, jnp.zeros((8 - aA, 4 * H), jnp.float32)],
                            axis=0)
                    dA = rdot(hA)
                    if aB > 0:
                        xb = lax.slice(xg, (s * bs + 8, 0),
                                       (s * bs + bs, 4 * H))
                        if aB < 8:
                            xb = jnp.concatenate(
                                [xb, jnp.zeros((8 - aB, 4 * H), jnp.float32)],
                                axis=0)
                        dB = rdot(hB)
                    ha_new, ca_new = acts(xa, dA, cA)
                    if aA < 8:
                        hA = jnp.where(mA, ha_new, hA)
                        cA = jnp.where(mA, ca_new, cA)
                    else:
                        hA, cA = ha_new, ca_new
                    if aB > 0:
                        hb_new, cb_new = acts(xb, dB, cB)
                        if aB < 8:
                            hB = jnp.where(mB, hb_new, hB)
                            cB = jnp.where(mB, cb_new, cB)
                        else:
                            hB, cB = hb_new, cb_new
                        outs[s] = jnp.concatenate(
                            [hA[0:aA, :], hB[0:aB, :]], axis=0)
                    else:
                        outs[s] = hA[0:aA, :]
                block = outs[0] if grp == 1 else jnp.concatenate(outs, axis=0)
                out_ref[pl.ds(goff, blk), :] = block
            return hA, cA, hB, cB

        hA, cA, hB, cB = lax.fori_loop(0, nit, gbody, (hA, cA, hB, cB))

    h_ref[...] = jnp.concatenate([hA, hB], axis=0)
    c_ref[...] = jnp.concatenate([cA, cB], axis=0)


def kernel(data, batch_sizes, h0, c0, W_ih, W_hh, b_ih, b_hh):
    del batch_sizes
    scale = jnp.concatenate([jnp.full((2 * H,), 0.5, jnp.float32),
                             jnp.ones((H,), jnp.float32),
                             jnp.full((H,), 0.5, jnp.float32)])
    W_ih = W_ih * scale[:, None]
    W_hh = W_hh * scale[:, None]
    b_ih = b_ih * scale
    b_hh = b_hh * scale
    out, h, c = pl.pallas_call(
        _rnn_kernel,
        out_shape=(
            jax.ShapeDtypeStruct((TOTAL, H), jnp.float32),
            jax.ShapeDtypeStruct((B, H), jnp.float32),
            jax.ShapeDtypeStruct((B, H), jnp.float32),
        ),
        in_specs=[pl.BlockSpec(memory_space=pltpu.VMEM)] * 7,
        out_specs=(
            pl.BlockSpec(memory_space=pltpu.VMEM),
            pl.BlockSpec(memory_space=pltpu.VMEM),
            pl.BlockSpec(memory_space=pltpu.VMEM),
        ),
        scratch_shapes=[pltpu.VMEM((TOTAL, 4 * H), jnp.float32)],
    )(data, h0, c0, W_ih.T, W_hh.T,
      b_ih.reshape(1, 4 * H), b_hh.reshape(1, 4 * H))
    return out, h, c


# speedup vs baseline: 1.1737x; 1.0460x over previous
import math

import jax
import jax.numpy as jnp
from jax import lax
from jax.experimental import pallas as pl
from jax.experimental.pallas import tpu as pltpu

D = 128
H = 128
B = 16
T = 512
STEPS = T // B
TOTAL = STEPS * (B * (B + 1)) // 2


def _rnn_kernel(data_ref, h0_ref, c0_ref, wih_t_ref, whh_t_ref,
                bih_ref, bhh_ref, out_ref, h_ref, c_ref, xp_ref):
    bias = bih_ref[...] + bhh_ref[...]
    wih_t = wih_t_ref[...]

    def proj_chunk(ci):
        off = TOTAL - 128 * (ci + 1)
        xp_ref[pl.ds(off, 128), :] = jnp.dot(
            data_ref[pl.ds(off, 128), :], wih_t,
            preferred_element_type=jnp.float32) + bias

    for ci in range(2):
        proj_chunk(ci)

    whh_t = whh_t_ref[...]
    hA = h0_ref[0:8, :]
    cA = c0_ref[0:8, :]
    hB = h0_ref[8:16, :]
    cB = c0_ref[8:16, :]

    def rdot(hb):
        return jnp.dot(hb, whh_t, preferred_element_type=jnp.float32)

    def acts(x, d, cb):
        raw = jnp.tanh(x + d)
        ig = 0.5 + 0.5 * raw[:, 0:H]
        fg = 0.5 + 0.5 * raw[:, H:2 * H]
        gg = raw[:, 2 * H:3 * H]
        og = 0.5 + 0.5 * raw[:, 3 * H:4 * H]
        c_new = fg * cb + ig * gg
        h_new = og * jnp.tanh(c_new)
        return h_new, c_new

    row = lax.broadcasted_iota(jnp.int32, (8, H), 0)

    for p in range(B):
        bs = p + 1
        aA = min(bs, 8)
        aB = bs - aA
        grp = math.lcm(bs, 8) // bs
        blk = grp * bs
        ngroups = STEPS // grp
        phase_end = TOTAL - STEPS * (p * (p + 1) // 2)
        mA = row < aA
        mB = row < aB

        ug = max(1, 32 // grp)
        nit = ngroups // ug

        def gbody(j, carry, bs=bs, aA=aA, aB=aB, grp=grp, blk=blk, ug=ug,
                  phase_end=phase_end, mA=mA, mB=mB, p=p):
            hA, cA, hB, cB = carry
            for t in range(ug):
                goff = pl.multiple_of(
                    phase_end - (j * ug + t + 1) * blk, 8)
                xg = xp_ref[pl.ds(goff, blk), :]
                outs = [None] * grp
                for m in range(grp):
                    s = grp - 1 - m
                    step = t * grp + m
                    if p < 4 and step % 4 == 0:
                        proj_chunk(2 + p * 8 + step // 4)
                    xa = lax.slice(xg, (s * bs, 0), (s * bs + aA, 4 * H))
                    if aA < 8:
                        xa = jnp.concatenate(
                            [xa, jnp.zeros((8 - aA, 4 * H), jnp.float32)],
                            axis=0)
                    dA = rdot(hA)
                    if aB > 0:
                        xb = lax.slice(xg, (s * bs + 8, 0),
                                       (s * bs + bs, 4 * H))
                        if aB < 8:
                            xb = jnp.concatenate(
                                [xb, jnp.zeros((8 - aB, 4 * H), jnp.float32)],
                                axis=0)
                        dB = rdot(hB)
                    ha_new, ca_new = acts(xa, dA, cA)
                    if aA < 8:
                        hA = jnp.where(mA, ha_new, hA)
                        cA = jnp.where(mA, ca_new, cA)
                    else:
                        hA, cA = ha_new, ca_new
                    if aB > 0:
                        hb_new, cb_new = acts(xb, dB, cB)
                        if aB < 8:
                            hB = jnp.where(mB, hb_new, hB)
                            cB = jnp.where(mB, cb_new, cB)
                        else:
                            hB, cB = hb_new, cb_new
                        outs[s] = jnp.concatenate(
                            [hA[0:aA, :], hB[0:aB, :]], axis=0)
                    else:
                        outs[s] = hA[0:aA, :]
                block = outs[0] if grp == 1 else jnp.concatenate(outs, axis=0)
                out_ref[pl.ds(goff, blk), :] = block
            return hA, cA, hB, cB

        hA, cA, hB, cB = lax.fori_loop(0, nit, gbody, (hA, cA, hB, cB))

    h_ref[...] = jnp.concatenate([hA, hB], axis=0)
    c_ref[...] = jnp.concatenate([cA, cB], axis=0)


def kernel(data, batch_sizes, h0, c0, W_ih, W_hh, b_ih, b_hh):
    del batch_sizes
    scale = jnp.concatenate([jnp.full((2 * H,), 0.5, jnp.float32),
                             jnp.ones((H,), jnp.float32),
                             jnp.full((H,), 0.5, jnp.float32)])
    W_ih = W_ih * scale[:, None]
    W_hh = W_hh * scale[:, None]
    b_ih = b_ih * scale
    b_hh = b_hh * scale
    out, h, c = pl.pallas_call(
        _rnn_kernel,
        out_shape=(
            jax.ShapeDtypeStruct((TOTAL, H), jnp.float32),
            jax.ShapeDtypeStruct((B, H), jnp.float32),
            jax.ShapeDtypeStruct((B, H), jnp.float32),
        ),
        in_specs=[pl.BlockSpec(memory_space=pltpu.VMEM)] * 7,
        out_specs=(
            pl.BlockSpec(memory_space=pltpu.VMEM),
            pl.BlockSpec(memory_space=pltpu.VMEM),
            pl.BlockSpec(memory_space=pltpu.VMEM),
        ),
        scratch_shapes=[pltpu.VMEM((TOTAL, 4 * H), jnp.float32)],
    )(data, h0, c0, W_ih.T, W_hh.T,
      b_ih.reshape(1, 4 * H), b_hh.reshape(1, 4 * H))
    return out, h, c


# speedup vs baseline: 1.1849x; 1.0096x over previous
import math

import jax
import jax.numpy as jnp
from jax import lax
from jax.experimental import pallas as pl
from jax.experimental.pallas import tpu as pltpu

D = 128
H = 128
B = 16
T = 512
STEPS = T // B
TOTAL = STEPS * (B * (B + 1)) // 2


def _rnn_kernel(data_ref, h0_ref, c0_ref, wih_t_ref, whh_t_ref,
                bih_ref, bhh_ref, out_ref, h_ref, c_ref, xp_ref):
    bias = bih_ref[...] + bhh_ref[...]
    wih_t = wih_t_ref[...]

    def proj_chunk(ci):
        off = TOTAL - 128 * (ci + 1)
        xp_ref[pl.ds(off, 128), :] = jnp.dot(
            data_ref[pl.ds(off, 128), :], wih_t,
            preferred_element_type=jnp.float32) + bias

    for ci in range(2):
        proj_chunk(ci)

    whh_t = whh_t_ref[...]
    hA = h0_ref[0:8, :]
    cA = c0_ref[0:8, :]
    hB = h0_ref[8:16, :]
    cB = c0_ref[8:16, :]

    def rdot(hb):
        return jnp.dot(hb, whh_t, preferred_element_type=jnp.float32)

    def acts(x, d, cb):
        raw = jnp.tanh(x + d)
        ig = 0.5 + 0.5 * raw[:, 0:H]
        fg = 0.5 + 0.5 * raw[:, H:2 * H]
        gg = raw[:, 2 * H:3 * H]
        og = 0.5 + 0.5 * raw[:, 3 * H:4 * H]
        c_new = fg * cb + ig * gg
        h_new = og * jnp.tanh(c_new)
        return h_new, c_new

    row = lax.broadcasted_iota(jnp.int32, (8, H), 0)
    row16 = lax.broadcasted_iota(jnp.int32, (B, H), 0)
    hS = jnp.concatenate([hA, hB], axis=0)
    cS = jnp.concatenate([cA, cB], axis=0)

    for p in range(B):
        bs = p + 1
        aA = min(bs, 8)
        aB = bs - aA
        grp = math.lcm(bs, 8) // bs
        blk = grp * bs
        ngroups = STEPS // grp
        phase_end = TOTAL - STEPS * (p * (p + 1) // 2)
        mA = row < aA
        mS = row16 < bs

        ug = max(1, 32 // grp)
        nit = ngroups // ug

        def gbody(j, carry, bs=bs, aA=aA, aB=aB, grp=grp, blk=blk, ug=ug,
                  phase_end=phase_end, mA=mA, mS=mS, p=p):
            if p < 8:
                hA, cA = carry
            else:
                hS, cS = carry
            for t in range(ug):
                goff = pl.multiple_of(
                    phase_end - (j * ug + t + 1) * blk, 8)
                xg = xp_ref[pl.ds(goff, blk), :]
                outs = [None] * grp
                for m in range(grp):
                    s = grp - 1 - m
                    step = t * grp + m
                    if p < 4 and step % 4 == 0:
                        proj_chunk(2 + p * 8 + step // 4)
                    if p < 8:
                        xa = lax.slice(xg, (s * bs, 0), (s * bs + aA, 4 * H))
                        if aA < 8:
                            xa = jnp.concatenate(
                                [xa,
                                 jnp.zeros((8 - aA, 4 * H), jnp.float32)],
                                axis=0)
                        ha_new, ca_new = acts(xa, rdot(hA), cA)
                        if aA < 8:
                            hA = jnp.where(mA, ha_new, hA)
                            cA = jnp.where(mA, ca_new, cA)
                        else:
                            hA, cA = ha_new, ca_new
                        outs[s] = hA[0:aA, :]
                    else:
                        xs = lax.slice(xg, (s * bs, 0), (s * bs + bs, 4 * H))
                        if bs < B:
                            xs = jnp.concatenate(
                                [xs,
                                 jnp.zeros((B - bs, 4 * H), jnp.float32)],
                                axis=0)
                        hs_new, cs_new = acts(xs, rdot(hS), cS)
                        if bs < B:
                            hS = jnp.where(mS, hs_new, hS)
                            cS = jnp.where(mS, cs_new, cS)
                        else:
                            hS, cS = hs_new, cs_new
                        outs[s] = hS[0:bs, :]
                block = outs[0] if grp == 1 else jnp.concatenate(outs, axis=0)
                out_ref[pl.ds(goff, blk), :] = block
            if p < 8:
                return hA, cA
            return hS, cS

        if p < 8:
            hA, cA = lax.fori_loop(0, nit, gbody, (hA, cA))
            if p == 7:
                hS = jnp.concatenate([hA, hB], axis=0)
                cS = jnp.concatenate([cA, cB], axis=0)
        else:
            hS, cS = lax.fori_loop(0, nit, gbody, (hS, cS))

    h_ref[...] = hS
    c_ref[...] = cS


def kernel(data, batch_sizes, h0, c0, W_ih, W_hh, b_ih, b_hh):
    del batch_sizes
    scale = jnp.concatenate([jnp.full((2 * H,), 0.5, jnp.float32),
                             jnp.ones((H,), jnp.float32),
                             jnp.full((H,), 0.5, jnp.float32)])
    W_ih = W_ih * scale[:, None]
    W_hh = W_hh * scale[:, None]
    b_ih = b_ih * scale
    b_hh = b_hh * scale
    out, h, c = pl.pallas_call(
        _rnn_kernel,
        out_shape=(
            jax.ShapeDtypeStruct((TOTAL, H), jnp.float32),
            jax.ShapeDtypeStruct((B, H), jnp.float32),
            jax.ShapeDtypeStruct((B, H), jnp.float32),
        ),
        in_specs=[pl.BlockSpec(memory_space=pltpu.VMEM)] * 7,
        out_specs=(
            pl.BlockSpec(memory_space=pltpu.VMEM),
            pl.BlockSpec(memory_space=pltpu.VMEM),
            pl.BlockSpec(memory_space=pltpu.VMEM),
        ),
        scratch_shapes=[pltpu.VMEM((TOTAL, 4 * H), jnp.float32)],
    )(data, h0, c0, W_ih.T, W_hh.T,
      b_ih.reshape(1, 4 * H), b_hh.reshape(1, 4 * H))
    return out, h, c
